# baseline (speedup 1.0000x reference)
import functools

import jax
import jax.numpy as jnp
from jax import lax
from jax.experimental import pallas as pl
from jax.experimental.pallas import tpu as pltpu
from jax.experimental.pallas import tpu_sc as plsc

B, N, D = 8, 2048, 128
NC, NS = 2, 16
NW = NC * NS
TOTAL = B * N
ROWS_PER_W = TOTAL // NW
CHUNK = 128
CHUNKS_PER_W = ROWS_PER_W // CHUNK
W_PER_B = N // ROWS_PER_W


def _sc_gather(table, vertices):
    mesh = plsc.VectorSubcoreMesh(core_axis_name="c", subcore_axis_name="s")

    @functools.partial(
        pl.kernel,
        mesh=mesh,
        out_type=jax.ShapeDtypeStruct((TOTAL, D), jnp.float32),
        scratch_types=[
            pltpu.VMEM((ROWS_PER_W,), jnp.int32),
            pltpu.VMEM((ROWS_PER_W, D), jnp.float32),
            pltpu.SemaphoreType.DMA,
            pltpu.SemaphoreType.DMA,
        ],
    )
    def k(table_hbm, vert_hbm, out_hbm, idx_v, rows_v, gsem, wsem):
        wid = lax.axis_index("s") * NC + lax.axis_index("c")
        b = wid // W_PER_B
        col0 = (wid % W_PER_B) * ROWS_PER_W
        pltpu.sync_copy(vert_hbm.at[b, pl.ds(col0, ROWS_PER_W)], idx_v)
        gathers = [
            pltpu.async_copy(
                table_hbm.at[idx_v.at[pl.ds(j * CHUNK, CHUNK)]],
                rows_v.at[pl.ds(j * CHUNK, CHUNK)],
                gsem,
            )
            for j in range(CHUNKS_PER_W)
        ]
        writes = []
        for j in range(CHUNKS_PER_W):
            gathers[j].wait()
            writes.append(
                pltpu.async_copy(
                    rows_v.at[pl.ds(j * CHUNK, CHUNK)],
                    out_hbm.at[pl.ds(wid * ROWS_PER_W + j * CHUNK, CHUNK)],
                    wsem,
                )
            )
        for w in writes:
            w.wait()

    return k(table, vertices)


def _lsm_half(v_ref, o_ref):
    v = v_ref[...]
    m = jnp.max(v, axis=1, keepdims=True)
    lse = m + jnp.log(jnp.sum(jnp.exp(v - m), axis=1, keepdims=True))
    o_ref[...] = v - lse


def _lsm_half2(v_ref, buf_ref, o_ref):
    del buf_ref
    _lsm_half(v_ref, o_ref)


def kernel(x, vertices, adj, emb_table):
    del adj
    emb = _sc_gather(emb_table, vertices.astype(jnp.int32)).reshape(B, N, D)

    buf = pl.pallas_call(
        _lsm_half,
        grid=(1,),
        in_specs=[pl.BlockSpec((8, N, D), lambda b: (b, 0, 0))],
        out_specs=pl.BlockSpec((8, N, D), lambda b: (b, 0, 0)),
        out_shape=jax.ShapeDtypeStruct((B, N, 2 * D), jnp.float32),
    )(x)

    out = pl.pallas_call(
        _lsm_half2,
        grid=(1,),
        in_specs=[
            pl.BlockSpec((8, N, D), lambda b: (b, 0, 0)),
            pl.BlockSpec(memory_space=pl.ANY),
        ],
        out_specs=pl.BlockSpec((8, N, D), lambda b: (b, 0, 1)),
        out_shape=jax.ShapeDtypeStruct((B, N, 2 * D), jnp.float32),
        input_output_aliases={1: 0},
    )(emb, buf)
    return out


# speedup vs baseline: 1.0373x; 1.0373x over previous
import functools

import jax
import jax.numpy as jnp
from jax import lax
from jax.experimental import pallas as pl
from jax.experimental.pallas import tpu as pltpu
from jax.experimental.pallas import tpu_sc as plsc

B, N, D = 8, 2048, 128
NC, NS = 2, 16
NW = NC * NS
TOTAL = B * N
ROWS_PER_W = TOTAL // NW
CHUNK = 128
CHUNKS_PER_W = ROWS_PER_W // CHUNK
W_PER_B = N // ROWS_PER_W


def _sc_gather(table, vertices):
    mesh = plsc.VectorSubcoreMesh(core_axis_name="c", subcore_axis_name="s")

    @functools.partial(
        pl.kernel,
        mesh=mesh,
        out_type=jax.ShapeDtypeStruct((TOTAL, D), jnp.float32),
        scratch_types=[
            pltpu.VMEM((ROWS_PER_W,), jnp.int32),
            pltpu.VMEM((ROWS_PER_W, D), jnp.float32),
            pltpu.SemaphoreType.DMA,
            pltpu.SemaphoreType.DMA,
        ],
    )
    def k(table_hbm, vert_hbm, out_hbm, idx_v, rows_v, gsem, wsem):
        wid = lax.axis_index("s") * NC + lax.axis_index("c")
        b = wid // W_PER_B
        col0 = (wid % W_PER_B) * ROWS_PER_W
        pltpu.sync_copy(vert_hbm.at[b, pl.ds(col0, ROWS_PER_W)], idx_v)
        gathers = [
            pltpu.async_copy(
                table_hbm.at[idx_v.at[pl.ds(j * CHUNK, CHUNK)]],
                rows_v.at[pl.ds(j * CHUNK, CHUNK)],
                gsem,
            )
            for j in range(CHUNKS_PER_W)
        ]
        writes = []
        for j in range(CHUNKS_PER_W):
            gathers[j].wait()
            writes.append(
                pltpu.async_copy(
                    rows_v.at[pl.ds(j * CHUNK, CHUNK)],
                    out_hbm.at[pl.ds(wid * ROWS_PER_W + j * CHUNK, CHUNK)],
                    wsem,
                )
            )
        for w in writes:
            w.wait()

    return k(table, vertices)


def _lsm_half(v_ref, o_ref):
    v = v_ref[...]
    m = jnp.max(v, axis=1, keepdims=True)
    lse = m + jnp.log(jnp.sum(jnp.exp(v - m), axis=1, keepdims=True))
    o_ref[...] = v - lse


def _lsm_half2(v_ref, buf_ref, o_ref):
    del buf_ref
    _lsm_half(v_ref, o_ref)


def kernel(x, vertices, adj, emb_table):
    del adj
    emb = _sc_gather(emb_table, vertices.astype(jnp.int32)).reshape(B, N, D)

    buf = pl.pallas_call(
        _lsm_half,
        grid=(B // 4,),
        in_specs=[pl.BlockSpec((4, N, D), lambda b: (b, 0, 0))],
        out_specs=pl.BlockSpec((4, N, D), lambda b: (b, 0, 0)),
        out_shape=jax.ShapeDtypeStruct((B, N, 2 * D), jnp.float32),
    )(x)

    out = pl.pallas_call(
        _lsm_half2,
        grid=(B // 4,),
        in_specs=[
            pl.BlockSpec((4, N, D), lambda b: (b, 0, 0)),
            pl.BlockSpec(memory_space=pl.ANY),
        ],
        out_specs=pl.BlockSpec((4, N, D), lambda b: (b, 0, 1)),
        out_shape=jax.ShapeDtypeStruct((B, N, 2 * D), jnp.float32),
        input_output_aliases={1: 0},
    )(emb, buf)
    return out
